# baseline (speedup 1.0000x reference)
import functools

import jax
import jax.numpy as jnp
from jax import lax
from jax.experimental import pallas as pl
from jax.experimental.pallas import tpu as pltpu
from jax.experimental.pallas import tpu_sc as plsc

NUM_CLASSES = 100000
IMAGE_SIZE = 16
NUM_CHANNELS = 3
BATCH = 16384
EMB_DIM = NUM_CHANNELS * IMAGE_SIZE * IMAGE_SIZE
CHAN = IMAGE_SIZE * IMAGE_SIZE

_NC = 2
_NS = 16
_NW = _NC * _NS
_ROWS_PER_W = BATCH // _NW
_CHUNK = 64
_NCHUNK = _ROWS_PER_W // _CHUNK
_NGRP = EMB_DIM // 16
_GPC = CHAN // 16

_BR = 512
_NBLK = BATCH // _BR


def _sc_gather_stats(label, table):
    mesh = plsc.VectorSubcoreMesh(core_axis_name="c", subcore_axis_name="s")

    @functools.partial(
        pl.kernel,
        mesh=mesh,
        out_type=(
            jax.ShapeDtypeStruct((BATCH, EMB_DIM), jnp.float32),
            jax.ShapeDtypeStruct((_NW, 6 * 16), jnp.float32),
        ),
        scratch_types=[
            pltpu.VMEM((_ROWS_PER_W,), jnp.int32),
            pltpu.VMEM((_CHUNK, EMB_DIM), jnp.float32),
            pltpu.VMEM((_CHUNK, EMB_DIM), jnp.float32),
            pltpu.VMEM((6 * 16,), jnp.float32),
            pltpu.SemaphoreType.DMA,
            pltpu.SemaphoreType.DMA,
            pltpu.SemaphoreType.DMA,
            pltpu.SemaphoreType.DMA,
        ],
    )
    def gather_kernel(label_hbm, table_hbm, out_hbm, parts_hbm,
                      idx_v, buf0, buf1, parts_v, g0, g1, s0, s1):
        wid = lax.axis_index("s") * _NC + lax.axis_index("c")
        base = wid * _ROWS_PER_W
        bufs = (buf0, buf1)
        gsems = (g0, g1)
        ssems = (s0, s1)

        pltpu.sync_copy(label_hbm.at[pl.ds(base, _ROWS_PER_W)], idx_v)

        accs = [jnp.zeros((16,), jnp.float32) for _ in range(6)]

        def chunk_stats(buf, carry):
            def row_body(r, carry_in):
                vals = list(carry_in)
                for j in range(_NGRP):
                    c = j // _GPC
                    v = buf[r, pl.ds(16 * j, 16)]
                    vals[c] = vals[c] + v
                    vals[3 + c] = vals[3 + c] + v * v
                return tuple(vals)

            return list(lax.fori_loop(0, _CHUNK, row_body, tuple(carry)))

        gh = [None, None]
        sh = [None, None]
        gh[0] = pltpu.async_copy(
            table_hbm.at[idx_v.at[pl.ds(0, _CHUNK)]], bufs[0], gsems[0])
        for c in range(_NCHUNK):
            cur = c & 1
            nxt = (c + 1) & 1
            gh[cur].wait()
            sh[cur] = pltpu.async_copy(
                bufs[cur], out_hbm.at[pl.ds(base + c * _CHUNK, _CHUNK)],
                ssems[cur])
            if c + 1 < _NCHUNK:
                if c >= 1:
                    sh[nxt].wait()
                gh[nxt] = pltpu.async_copy(
                    table_hbm.at[idx_v.at[pl.ds((c + 1) * _CHUNK, _CHUNK)]],
                    bufs[nxt], gsems[nxt])
            accs = chunk_stats(bufs[cur], accs)
        sh[(_NCHUNK - 2) & 1].wait()
        sh[(_NCHUNK - 1) & 1].wait()

        for c in range(6):
            parts_v[pl.ds(16 * c, 16)] = accs[c]
        pltpu.sync_copy(parts_v, parts_hbm.at[wid])

    return gather_kernel(label, table)


def _norm_body(parts_ref, w_ref, b_ref, x_ref, o_ref, par_ref):
    i = pl.program_id(0)

    @pl.when(i == 0)
    def _params():
        p = parts_ref[...]
        n = jnp.float32(BATCH * CHAN)
        for c in range(NUM_CHANNELS):
            s = jnp.sum(p[:, 16 * c : 16 * c + 16])
            q = jnp.sum(p[:, 48 + 16 * c : 48 + 16 * c + 16])
            mean = s / n
            var = q / n - mean * mean
            scale = lax.rsqrt(var + 1e-5) * w_ref[c]
            par_ref[2 * c] = scale
            par_ref[2 * c + 1] = b_ref[c] - mean * scale

    for c in range(NUM_CHANNELS):
        o_ref[:, c, :] = (
            x_ref[:, CHAN * c : CHAN * (c + 1)] * par_ref[2 * c]
            + par_ref[2 * c + 1]
        )


def kernel(label, table, bn_weight, bn_bias):
    gathered, parts = _sc_gather_stats(label, table)

    out = pl.pallas_call(
        _norm_body,
        grid=(_NBLK,),
        in_specs=[
            pl.BlockSpec((_NW, 6 * 16), lambda i: (0, 0)),
            pl.BlockSpec(memory_space=pltpu.SMEM),
            pl.BlockSpec(memory_space=pltpu.SMEM),
            pl.BlockSpec((_BR, EMB_DIM), lambda i: (i, 0)),
        ],
        out_specs=pl.BlockSpec((_BR, NUM_CHANNELS, CHAN), lambda i: (i, 0, 0)),
        out_shape=jax.ShapeDtypeStruct((BATCH, NUM_CHANNELS, CHAN), jnp.float32),
        scratch_shapes=[pltpu.SMEM((8,), jnp.float32)],
    )(parts, bn_weight, bn_bias, gathered)

    return out.reshape(-1, NUM_CHANNELS, IMAGE_SIZE, IMAGE_SIZE)


# speedup vs baseline: 1.6978x; 1.6978x over previous
import functools

import jax
import jax.numpy as jnp
from jax import lax
from jax.experimental import pallas as pl
from jax.experimental.pallas import tpu as pltpu
from jax.experimental.pallas import tpu_sc as plsc

NUM_CLASSES = 100000
IMAGE_SIZE = 16
NUM_CHANNELS = 3
BATCH = 16384
EMB_DIM = NUM_CHANNELS * IMAGE_SIZE * IMAGE_SIZE
CHAN = IMAGE_SIZE * IMAGE_SIZE

_NC = 2
_NS = 16
_NW = _NC * _NS
_ROWS_PER_W = BATCH // _NW
_CHUNK = 64
_NCHUNK = _ROWS_PER_W // _CHUNK
_NGRP = EMB_DIM // 16
_GPC = CHAN // 16

_BR = 512
_NBLK = BATCH // _BR


def _sc_gather_stats(label, table):
    mesh = plsc.VectorSubcoreMesh(core_axis_name="c", subcore_axis_name="s")

    @functools.partial(
        pl.kernel,
        mesh=mesh,
        out_type=(
            jax.ShapeDtypeStruct((BATCH, EMB_DIM), jnp.float32),
            jax.ShapeDtypeStruct((_NW, 6 * 16), jnp.float32),
        ),
        scratch_types=[
            pltpu.VMEM((_ROWS_PER_W,), jnp.int32),
            pltpu.VMEM((_CHUNK, EMB_DIM), jnp.float32),
            pltpu.VMEM((_CHUNK, EMB_DIM), jnp.float32),
            pltpu.VMEM((6 * 16,), jnp.float32),
            pltpu.SemaphoreType.DMA,
            pltpu.SemaphoreType.DMA,
            pltpu.SemaphoreType.DMA,
            pltpu.SemaphoreType.DMA,
        ],
    )
    def gather_kernel(label_hbm, table_hbm, out_hbm, parts_hbm,
                      idx_v, buf0, buf1, parts_v, g0, g1, s0, s1):
        wid = lax.axis_index("s") * _NC + lax.axis_index("c")
        base = wid * _ROWS_PER_W
        bufs = (buf0, buf1)
        gsems = (g0, g1)
        ssems = (s0, s1)

        pltpu.sync_copy(label_hbm.at[pl.ds(base, _ROWS_PER_W)], idx_v)

        accs = [jnp.zeros((16,), jnp.float32) for _ in range(6)]

        def chunk_stats(buf, carry):
            def row_body(r, carry_in):
                vals = list(carry_in)
                for j in range(_NGRP):
                    c = j // _GPC
                    v = buf[r, pl.ds(16 * j, 16)]
                    vals[c] = vals[c] + v
                    vals[3 + c] = vals[3 + c] + v * v
                return tuple(vals)

            return list(lax.fori_loop(0, _CHUNK, row_body, tuple(carry)))

        gh = [None, None]
        sh = [None, None]
        gh[0] = pltpu.async_copy(
            table_hbm.at[idx_v.at[pl.ds(0, _CHUNK)]], bufs[0], gsems[0])
        for c in range(_NCHUNK):
            cur = c & 1
            nxt = (c + 1) & 1
            gh[cur].wait()
            sh[cur] = pltpu.async_copy(
                bufs[cur], out_hbm.at[pl.ds(base + c * _CHUNK, _CHUNK)],
                ssems[cur])
            if c + 1 < _NCHUNK:
                if c >= 1:
                    sh[nxt].wait()
                gh[nxt] = pltpu.async_copy(
                    table_hbm.at[idx_v.at[pl.ds((c + 1) * _CHUNK, _CHUNK)]],
                    bufs[nxt], gsems[nxt])
            accs = chunk_stats(bufs[cur], accs)
        sh[(_NCHUNK - 2) & 1].wait()
        sh[(_NCHUNK - 1) & 1].wait()

        for c in range(6):
            parts_v[pl.ds(16 * c, 16)] = accs[c]
        pltpu.sync_copy(parts_v, parts_hbm.at[wid])

    return gather_kernel(label, table)


def _norm_body(parts_ref, w_ref, b_ref, x_ref, o_ref, par_ref):
    i = pl.program_id(0)

    @pl.when(i == 0)
    def _params():
        p = parts_ref[...]
        n = jnp.float32(BATCH * CHAN)
        for c in range(NUM_CHANNELS):
            s = jnp.sum(p[:, 16 * c : 16 * c + 16])
            q = jnp.sum(p[:, 48 + 16 * c : 48 + 16 * c + 16])
            mean = s / n
            var = q / n - mean * mean
            scale = lax.rsqrt(var + 1e-5) * w_ref[c]
            par_ref[2 * c] = scale
            par_ref[2 * c + 1] = b_ref[c] - mean * scale

    ys = [
        x_ref[:, CHAN * c : CHAN * (c + 1)] * par_ref[2 * c] + par_ref[2 * c + 1]
        for c in range(NUM_CHANNELS)
    ]
    y = jnp.concatenate(ys, axis=1)
    o_ref[...] = y.T


def kernel(label, table, bn_weight, bn_bias):
    gathered, parts = _sc_gather_stats(label, table)

    out_t = pl.pallas_call(
        _norm_body,
        grid=(_NBLK,),
        in_specs=[
            pl.BlockSpec((_NW, 6 * 16), lambda i: (0, 0)),
            pl.BlockSpec(memory_space=pltpu.SMEM),
            pl.BlockSpec(memory_space=pltpu.SMEM),
            pl.BlockSpec((_BR, EMB_DIM), lambda i: (i, 0)),
        ],
        out_specs=pl.BlockSpec((EMB_DIM, _BR), lambda i: (0, i)),
        out_shape=jax.ShapeDtypeStruct((EMB_DIM, BATCH), jnp.float32),
        scratch_shapes=[pltpu.SMEM((8,), jnp.float32)],
    )(parts, bn_weight, bn_bias, gathered)

    return out_t.T.reshape(-1, NUM_CHANNELS, IMAGE_SIZE, IMAGE_SIZE)


# speedup vs baseline: 1.8764x; 1.1052x over previous
import functools

import jax
import jax.numpy as jnp
from jax import lax
from jax.experimental import pallas as pl
from jax.experimental.pallas import tpu as pltpu
from jax.experimental.pallas import tpu_sc as plsc

NUM_CLASSES = 100000
IMAGE_SIZE = 16
NUM_CHANNELS = 3
BATCH = 16384
EMB_DIM = NUM_CHANNELS * IMAGE_SIZE * IMAGE_SIZE
CHAN = IMAGE_SIZE * IMAGE_SIZE

_NC = 2
_NS = 16
_NW = _NC * _NS
_ROWS_PER_W = BATCH // _NW
_CHUNK = 64
_NCHUNK = _ROWS_PER_W // _CHUNK
_NGRP = EMB_DIM // 16
_GPC = CHAN // 16

_BR = 2048
_NBLK = BATCH // _BR


def _sc_gather_stats(label, table):
    mesh = plsc.VectorSubcoreMesh(core_axis_name="c", subcore_axis_name="s")

    @functools.partial(
        pl.kernel,
        mesh=mesh,
        out_type=(
            jax.ShapeDtypeStruct((BATCH, EMB_DIM), jnp.float32),
            jax.ShapeDtypeStruct((_NW, 6 * 16), jnp.float32),
        ),
        scratch_types=[
            pltpu.VMEM((_ROWS_PER_W,), jnp.int32),
            pltpu.VMEM((_CHUNK, EMB_DIM), jnp.float32),
            pltpu.VMEM((_CHUNK, EMB_DIM), jnp.float32),
            pltpu.VMEM((6 * 16,), jnp.float32),
            pltpu.SemaphoreType.DMA,
            pltpu.SemaphoreType.DMA,
            pltpu.SemaphoreType.DMA,
            pltpu.SemaphoreType.DMA,
        ],
    )
    def gather_kernel(label_hbm, table_hbm, out_hbm, parts_hbm,
                      idx_v, buf0, buf1, parts_v, g0, g1, s0, s1):
        wid = lax.axis_index("s") * _NC + lax.axis_index("c")
        base = wid * _ROWS_PER_W
        bufs = (buf0, buf1)
        gsems = (g0, g1)
        ssems = (s0, s1)

        pltpu.sync_copy(label_hbm.at[pl.ds(base, _ROWS_PER_W)], idx_v)

        accs = [jnp.zeros((16,), jnp.float32) for _ in range(6)]

        def chunk_stats(buf, carry):
            def row_body(r, carry_in):
                vals = list(carry_in)
                for j in range(_NGRP):
                    c = j // _GPC
                    v = buf[r, pl.ds(16 * j, 16)]
                    vals[c] = vals[c] + v
                    vals[3 + c] = vals[3 + c] + v * v
                return tuple(vals)

            return list(lax.fori_loop(0, _CHUNK, row_body, tuple(carry)))

        gh = [None, None]
        sh = [None, None]
        gh[0] = pltpu.async_copy(
            table_hbm.at[idx_v.at[pl.ds(0, _CHUNK)]], bufs[0], gsems[0])
        for c in range(_NCHUNK):
            cur = c & 1
            nxt = (c + 1) & 1
            gh[cur].wait()
            sh[cur] = pltpu.async_copy(
                bufs[cur], out_hbm.at[pl.ds(base + c * _CHUNK, _CHUNK)],
                ssems[cur])
            if c + 1 < _NCHUNK:
                if c >= 1:
                    sh[nxt].wait()
                gh[nxt] = pltpu.async_copy(
                    table_hbm.at[idx_v.at[pl.ds((c + 1) * _CHUNK, _CHUNK)]],
                    bufs[nxt], gsems[nxt])
            accs = chunk_stats(bufs[cur], accs)
        sh[(_NCHUNK - 2) & 1].wait()
        sh[(_NCHUNK - 1) & 1].wait()

        for c in range(6):
            parts_v[pl.ds(16 * c, 16)] = accs[c]
        pltpu.sync_copy(parts_v, parts_hbm.at[wid])

    return gather_kernel(label, table)


def _norm_body(parts_ref, w_ref, b_ref, x_ref, o_ref, par_ref):
    i = pl.program_id(0)

    @pl.when(i == 0)
    def _params():
        p = parts_ref[...]
        n = jnp.float32(BATCH * CHAN)
        for c in range(NUM_CHANNELS):
            s = jnp.sum(p[:, 16 * c : 16 * c + 16])
            q = jnp.sum(p[:, 48 + 16 * c : 48 + 16 * c + 16])
            mean = s / n
            var = q / n - mean * mean
            scale = lax.rsqrt(var + 1e-5) * w_ref[c]
            par_ref[2 * c] = scale
            par_ref[2 * c + 1] = b_ref[c] - mean * scale

    ys = [
        x_ref[:, CHAN * c : CHAN * (c + 1)] * par_ref[2 * c] + par_ref[2 * c + 1]
        for c in range(NUM_CHANNELS)
    ]
    y = jnp.concatenate(ys, axis=1)
    o_ref[...] = y.T


def kernel(label, table, bn_weight, bn_bias):
    gathered, parts = _sc_gather_stats(label, table)

    out_t = pl.pallas_call(
        _norm_body,
        grid=(_NBLK,),
        in_specs=[
            pl.BlockSpec((_NW, 6 * 16), lambda i: (0, 0)),
            pl.BlockSpec(memory_space=pltpu.SMEM),
            pl.BlockSpec(memory_space=pltpu.SMEM),
            pl.BlockSpec((_BR, EMB_DIM), lambda i: (i, 0)),
        ],
        out_specs=pl.BlockSpec((EMB_DIM, _BR), lambda i: (0, i)),
        out_shape=jax.ShapeDtypeStruct((EMB_DIM, BATCH), jnp.float32),
        scratch_shapes=[pltpu.SMEM((8,), jnp.float32)],
    )(parts, bn_weight, bn_bias, gathered)

    return out_t.T.reshape(-1, NUM_CHANNELS, IMAGE_SIZE, IMAGE_SIZE)


# speedup vs baseline: 1.8883x; 1.0064x over previous
import functools

import jax
import jax.numpy as jnp
from jax import lax
from jax.experimental import pallas as pl
from jax.experimental.pallas import tpu as pltpu
from jax.experimental.pallas import tpu_sc as plsc

NUM_CLASSES = 100000
IMAGE_SIZE = 16
NUM_CHANNELS = 3
BATCH = 16384
EMB_DIM = NUM_CHANNELS * IMAGE_SIZE * IMAGE_SIZE
CHAN = IMAGE_SIZE * IMAGE_SIZE

_NC = 2
_NS = 16
_NW = _NC * _NS
_ROWS_PER_W = BATCH // _NW
_CHUNK = 64
_NCHUNK = _ROWS_PER_W // _CHUNK
_NGRP = EMB_DIM // 16
_GPC = CHAN // 16

_BR = 4096
_NBLK = BATCH // _BR


def _sc_gather_stats(label, table):
    mesh = plsc.VectorSubcoreMesh(core_axis_name="c", subcore_axis_name="s")

    @functools.partial(
        pl.kernel,
        mesh=mesh,
        out_type=(
            jax.ShapeDtypeStruct((BATCH, EMB_DIM), jnp.float32),
            jax.ShapeDtypeStruct((_NW, 6 * 16), jnp.float32),
        ),
        scratch_types=[
            pltpu.VMEM((_ROWS_PER_W,), jnp.int32),
            pltpu.VMEM((_CHUNK, EMB_DIM), jnp.float32),
            pltpu.VMEM((_CHUNK, EMB_DIM), jnp.float32),
            pltpu.VMEM((6 * 16,), jnp.float32),
            pltpu.SemaphoreType.DMA,
            pltpu.SemaphoreType.DMA,
            pltpu.SemaphoreType.DMA,
            pltpu.SemaphoreType.DMA,
        ],
    )
    def gather_kernel(label_hbm, table_hbm, out_hbm, parts_hbm,
                      idx_v, buf0, buf1, parts_v, g0, g1, s0, s1):
        wid = lax.axis_index("s") * _NC + lax.axis_index("c")
        base = wid * _ROWS_PER_W
        bufs = (buf0, buf1)
        gsems = (g0, g1)
        ssems = (s0, s1)

        pltpu.sync_copy(label_hbm.at[pl.ds(base, _ROWS_PER_W)], idx_v)

        accs = [jnp.zeros((16,), jnp.float32) for _ in range(6)]

        def chunk_stats(buf, carry):
            def row_body(r, carry_in):
                vals = list(carry_in)
                for j in range(_NGRP):
                    c = j // _GPC
                    v = buf[r, pl.ds(16 * j, 16)]
                    vals[c] = vals[c] + v
                    vals[3 + c] = vals[3 + c] + v * v
                return tuple(vals)

            return list(lax.fori_loop(0, _CHUNK, row_body, tuple(carry)))

        gh = [None, None]
        sh = [None, None]
        gh[0] = pltpu.async_copy(
            table_hbm.at[idx_v.at[pl.ds(0, _CHUNK)]], bufs[0], gsems[0])
        for c in range(_NCHUNK):
            cur = c & 1
            nxt = (c + 1) & 1
            gh[cur].wait()
            sh[cur] = pltpu.async_copy(
                bufs[cur], out_hbm.at[pl.ds(base + c * _CHUNK, _CHUNK)],
                ssems[cur])
            if c + 1 < _NCHUNK:
                if c >= 1:
                    sh[nxt].wait()
                gh[nxt] = pltpu.async_copy(
                    table_hbm.at[idx_v.at[pl.ds((c + 1) * _CHUNK, _CHUNK)]],
                    bufs[nxt], gsems[nxt])
            accs = chunk_stats(bufs[cur], accs)
        sh[(_NCHUNK - 2) & 1].wait()
        sh[(_NCHUNK - 1) & 1].wait()

        for c in range(6):
            parts_v[pl.ds(16 * c, 16)] = accs[c]
        pltpu.sync_copy(parts_v, parts_hbm.at[wid])

    return gather_kernel(label, table)


def _norm_body(parts_ref, w_ref, b_ref, x_ref, o_ref, par_ref):
    i = pl.program_id(0)

    @pl.when(i == 0)
    def _params():
        p = parts_ref[...]
        n = jnp.float32(BATCH * CHAN)
        for c in range(NUM_CHANNELS):
            s = jnp.sum(p[:, 16 * c : 16 * c + 16])
            q = jnp.sum(p[:, 48 + 16 * c : 48 + 16 * c + 16])
            mean = s / n
            var = q / n - mean * mean
            scale = lax.rsqrt(var + 1e-5) * w_ref[c]
            par_ref[2 * c] = scale
            par_ref[2 * c + 1] = b_ref[c] - mean * scale

    ys = [
        x_ref[:, CHAN * c : CHAN * (c + 1)] * par_ref[2 * c] + par_ref[2 * c + 1]
        for c in range(NUM_CHANNELS)
    ]
    y = jnp.concatenate(ys, axis=1)
    o_ref[...] = y.T


def kernel(label, table, bn_weight, bn_bias):
    gathered, parts = _sc_gather_stats(label, table)

    out_t = pl.pallas_call(
        _norm_body,
        grid=(_NBLK,),
        in_specs=[
            pl.BlockSpec((_NW, 6 * 16), lambda i: (0, 0)),
            pl.BlockSpec(memory_space=pltpu.SMEM),
            pl.BlockSpec(memory_space=pltpu.SMEM),
            pl.BlockSpec((_BR, EMB_DIM), lambda i: (i, 0)),
        ],
        out_specs=pl.BlockSpec((EMB_DIM, _BR), lambda i: (0, i)),
        out_shape=jax.ShapeDtypeStruct((EMB_DIM, BATCH), jnp.float32),
        scratch_shapes=[pltpu.SMEM((8,), jnp.float32)],
    )(parts, bn_weight, bn_bias, gathered)

    return out_t.T.reshape(-1, NUM_CHANNELS, IMAGE_SIZE, IMAGE_SIZE)
